# Initial kernel scaffold; baseline (speedup 1.0000x reference)
import jax
import jax.numpy as jnp
from jax.experimental import pallas as pl


def kernel(inputs_embed, labels_embed, labels, all_labels_embed, all_labels):
    raise NotImplementedError("write your pallas kernel here")


# speedup vs baseline: 10.4660x; 10.4660x over previous
import functools

import jax
import jax.numpy as jnp
import numpy as np
from jax import lax
from jax.experimental import pallas as pl
from jax.experimental.pallas import tpu as pltpu
from jax.experimental.pallas import tpu_sc as plsc

NUM_NEG = 50
NEG_INF = -1e9
_B = 1024
_D = 128
_NL = 1000
_JPAD = 64
_W = 4 * _JPAD

def _neg_ids():
    ka, kb = jax.random.split(jax.random.key(42))
    ids_a = jax.random.randint(ka, (_B, NUM_NEG), 0, _B)
    ids_b = jax.random.randint(kb, (_B, NUM_NEG), 0, _NL)
    pad = jnp.zeros((_B, _JPAD - NUM_NEG), jnp.int32)
    return (jnp.concatenate([ids_a.astype(jnp.int32), pad], axis=1),
            jnp.concatenate([ids_b.astype(jnp.int32), pad], axis=1))

_RB = 256


def _mm_body(i_blk, l_blk, it_full, bt_full, gib, glb, gii, gli, sp):
    ib = i_blk[...]
    lb = l_blk[...]
    it = it_full[...]
    bt = bt_full[...]
    dot = functools.partial(
        lax.dot_general,
        dimension_numbers=(((1,), (0,)), ((), ())),
        precision=lax.Precision.HIGHEST,
        preferred_element_type=jnp.float32,
    )
    gib[...] = dot(ib, bt)
    glb[...] = dot(lb, bt)
    gii[...] = dot(ib, it)
    gli[...] = dot(lb, it)
    sp[...] = jnp.sum(ib * lb, axis=1, keepdims=True)


_mm_call = pl.pallas_call(
    _mm_body,
    grid=(_B // _RB,),
    in_specs=[
        pl.BlockSpec((_RB, _D), lambda g: (g, 0)),
        pl.BlockSpec((_RB, _D), lambda g: (g, 0)),
        pl.BlockSpec((_D, _B), lambda g: (0, 0)),
        pl.BlockSpec((_D, _B), lambda g: (0, 0)),
    ],
    out_specs=[
        pl.BlockSpec((_RB, _B), lambda g: (g, 0)),
        pl.BlockSpec((_RB, _B), lambda g: (g, 0)),
        pl.BlockSpec((_RB, _B), lambda g: (g, 0)),
        pl.BlockSpec((_RB, _B), lambda g: (g, 0)),
        pl.BlockSpec((_RB, 1), lambda g: (g, 0)),
    ],
    out_shape=[
        jax.ShapeDtypeStruct((_B, _B), jnp.float32),
        jax.ShapeDtypeStruct((_B, _B), jnp.float32),
        jax.ShapeDtypeStruct((_B, _B), jnp.float32),
        jax.ShapeDtypeStruct((_B, _B), jnp.float32),
        jax.ShapeDtypeStruct((_B, 1), jnp.float32),
    ],
)

_NW = 32
_RPW = _B // _NW
_C = 8
_NCH = _RPW // _C


def _sc_body(gib, glb, gii, gli, labels, alabels, ids_a, ids_b, out,
             gib_v, glb_v, gii_v, gli_v, lab_v, alab_v, ia_v, ib_v, out_v):
    wid = lax.axis_index("c") * 16 + lax.axis_index("s")
    base = wid * _RPW
    pltpu.sync_copy(labels, lab_v)
    pltpu.sync_copy(alabels, alab_v)
    for t in range(_NCH):
        row0 = base + t * _C
        pltpu.sync_copy(gib.at[pl.ds(row0, _C)], gib_v)
        pltpu.sync_copy(glb.at[pl.ds(row0, _C)], glb_v)
        pltpu.sync_copy(gii.at[pl.ds(row0, _C)], gii_v)
        pltpu.sync_copy(gli.at[pl.ds(row0, _C)], gli_v)
        pltpu.sync_copy(ids_a.at[pl.ds(row0, _C)], ia_v)
        pltpu.sync_copy(ids_b.at[pl.ds(row0, _C)], ib_v)
        zero16 = jnp.zeros((16,), jnp.int32)
        for r in range(_C):
            rsplat = jnp.full((16,), r, jnp.int32)
            lab_i = plsc.load_gather(
                lab_v, [zero16, jnp.full((16,), row0 + r, jnp.int32)])
            for c in range(_JPAD // 16):
                s = c * 16
                jb = ib_v[r, pl.ds(s, 16)]
                ja = ia_v[r, pl.ds(s, 16)]
                pen_b = jnp.where(
                    plsc.load_gather(alab_v, [zero16, jb]) == lab_i,
                    NEG_INF, 0.0)
                pen_a = jnp.where(
                    plsc.load_gather(lab_v, [zero16, ja]) == lab_i,
                    NEG_INF, 0.0)
                out_v[r, pl.ds(s, 16)] = (
                    plsc.load_gather(gib_v, [rsplat, jb]) + pen_b)
                out_v[r, pl.ds(_JPAD + s, 16)] = (
                    plsc.load_gather(gli_v, [rsplat, ja]) + pen_a)
                out_v[r, pl.ds(2 * _JPAD + s, 16)] = (
                    plsc.load_gather(glb_v, [rsplat, jb]) + pen_b)
                out_v[r, pl.ds(3 * _JPAD + s, 16)] = (
                    plsc.load_gather(gii_v, [rsplat, ja]) + pen_a)
        pltpu.sync_copy(out_v, out.at[pl.ds(row0, _C)])


@functools.lru_cache(maxsize=1)
def _sc_gather():
    return pl.kernel(
        _sc_body,
        out_type=jax.ShapeDtypeStruct((_B, _W), jnp.float32),
        mesh=plsc.VectorSubcoreMesh(core_axis_name="c", subcore_axis_name="s"),
        compiler_params=pltpu.CompilerParams(needs_layout_passes=False),
        scratch_types=[
            pltpu.VMEM((_C, _B), jnp.float32),
            pltpu.VMEM((_C, _B), jnp.float32),
            pltpu.VMEM((_C, _B), jnp.float32),
            pltpu.VMEM((_C, _B), jnp.float32),
            pltpu.VMEM((1, _B), jnp.float32),
            pltpu.VMEM((1, _B), jnp.float32),
            pltpu.VMEM((_C, _JPAD), jnp.int32),
            pltpu.VMEM((_C, _JPAD), jnp.int32),
            pltpu.VMEM((_C, _W), jnp.float32),
        ],
    )


def _loss_body(sims_ref, sp_ref, loss_ref, acc_ref):
    x = sims_ref[...]
    sp = sp_ref[...][:, 0]
    col = lax.broadcasted_iota(jnp.int32, (_B, _W), 1)
    jj = col % _JPAD
    seg = col // _JPAD
    valid = jj < NUM_NEG
    xm = jnp.where(valid, x, NEG_INF)
    softm = valid & (seg < 2)
    xs = jnp.where(softm, xm, NEG_INF)
    m = jnp.maximum(jnp.max(xs, axis=1), sp)
    ssum = (jnp.sum(jnp.where(softm, jnp.exp(xs - m[:, None]), 0.0), axis=1)
            + jnp.exp(sp - m))
    softmax_loss = m + jnp.log(ssum) - sp
    ce_neg = jnp.where(
        valid, jnp.maximum(xm, 0.0) + jnp.log1p(jnp.exp(-jnp.abs(xm))), 0.0)
    ce_pos = jnp.maximum(sp, 0.0) - sp + jnp.log1p(jnp.exp(-jnp.abs(sp)))
    sigmoid_loss = (jnp.sum(ce_neg, axis=1) + ce_pos) / (4 * NUM_NEG + 1)
    negmax = jnp.max(jnp.where(valid & (seg == 0), xm, NEG_INF), axis=1)
    sim_max = jnp.maximum(sp, negmax)
    acc_ref[...] = jnp.mean((sim_max == sp).astype(jnp.float32)).reshape(1, 1)
    loss_ref[...] = jnp.mean(softmax_loss + sigmoid_loss).reshape(1, 1)


_loss_call = pl.pallas_call(
    _loss_body,
    out_shape=[
        jax.ShapeDtypeStruct((1, 1), jnp.float32),
        jax.ShapeDtypeStruct((1, 1), jnp.float32),
    ],
)


def kernel(inputs_embed, labels_embed, labels, all_labels_embed, all_labels):
    i = inputs_embed.astype(jnp.float32)
    l = labels_embed.astype(jnp.float32)
    it = i.T
    bt = jnp.zeros((_D, _B), jnp.float32).at[:, :_NL].set(all_labels_embed.T)
    gib, glb, gii, gli, sp = _mm_call(i, l, it, bt)
    lab1 = labels[:, 0].astype(jnp.float32).reshape(1, _B)
    alab1 = jnp.concatenate(
        [all_labels[:, 0].astype(jnp.float32),
         jnp.full((_B - _NL,), -1.0, jnp.float32)]).reshape(1, _B)
    ids_a, ids_b = _neg_ids()
    sims = _sc_gather()(gib, glb, gii, gli, lab1, alab1, ids_a, ids_b)
    loss, acc = _loss_call(sims, sp)
    return loss[0, 0], acc[0, 0]


# speedup vs baseline: 12.6705x; 1.2106x over previous
import functools

import jax
import jax.numpy as jnp
import numpy as np
from jax import lax
from jax.experimental import pallas as pl
from jax.experimental.pallas import tpu as pltpu
from jax.experimental.pallas import tpu_sc as plsc

NUM_NEG = 50
NEG_INF = -1e9
_B = 1024
_D = 128
_NL = 1000
_JPAD = 64
_W = 4 * _JPAD

def _neg_ids():
    ka, kb = jax.random.split(jax.random.key(42))
    ids_a = jax.random.randint(ka, (_B, NUM_NEG), 0, _B)
    ids_b = jax.random.randint(kb, (_B, NUM_NEG), 0, _NL)
    pad = jnp.zeros((_B, _JPAD - NUM_NEG), jnp.int32)
    return (jnp.concatenate([ids_a.astype(jnp.int32), pad], axis=1),
            jnp.concatenate([ids_b.astype(jnp.int32), pad], axis=1))

_RB = 256


def _mm_body(i_blk, l_blk, i_full, b_full, gib, glb, gii, gli, sp):
    ib = i_blk[...]
    lb = l_blk[...]
    it = i_full[...]
    bt = b_full[...]
    dot = functools.partial(
        lax.dot_general,
        dimension_numbers=(((1,), (1,)), ((), ())),
        precision=lax.Precision.HIGHEST,
        preferred_element_type=jnp.float32,
    )
    gib[...] = dot(ib, bt)
    glb[...] = dot(lb, bt)
    gii[...] = dot(ib, it)
    gli[...] = dot(lb, it)
    sp[...] = jnp.sum(ib * lb, axis=1, keepdims=True)


_mm_call = pl.pallas_call(
    _mm_body,
    grid=(_B // _RB,),
    in_specs=[
        pl.BlockSpec((_RB, _D), lambda g: (g, 0)),
        pl.BlockSpec((_RB, _D), lambda g: (g, 0)),
        pl.BlockSpec((_B, _D), lambda g: (0, 0)),
        pl.BlockSpec((_B, _D), lambda g: (0, 0)),
    ],
    out_specs=[
        pl.BlockSpec((_RB, _B), lambda g: (g, 0)),
        pl.BlockSpec((_RB, _B), lambda g: (g, 0)),
        pl.BlockSpec((_RB, _B), lambda g: (g, 0)),
        pl.BlockSpec((_RB, _B), lambda g: (g, 0)),
        pl.BlockSpec((_RB, 1), lambda g: (g, 0)),
    ],
    out_shape=[
        jax.ShapeDtypeStruct((_B, _B), jnp.float32),
        jax.ShapeDtypeStruct((_B, _B), jnp.float32),
        jax.ShapeDtypeStruct((_B, _B), jnp.float32),
        jax.ShapeDtypeStruct((_B, _B), jnp.float32),
        jax.ShapeDtypeStruct((_B, 1), jnp.float32),
    ],
)

_NW = 32
_RPW = _B // _NW
_C = 8
_NCH = _RPW // _C


def _sc_body(gib, glb, gii, gli, labels, alabels, ids_a, ids_b, out,
             gib0, glb0, gii0, gli0, ia0, ib0, out0,
             gib1, glb1, gii1, gli1, ia1, ib1, out1,
             lab_v, alab_v, ld0, ld1, st0, st1):
    wid = lax.axis_index("c") * 16 + lax.axis_index("s")
    base = wid * _RPW
    bufs = [(gib0, glb0, gii0, gli0, ia0, ib0, out0, ld0, st0),
            (gib1, glb1, gii1, gli1, ia1, ib1, out1, ld1, st1)]

    def chunk_pairs(t):
        row0 = base + t * _C
        gv = bufs[t % 2]
        pairs = [(gib.at[pl.ds(row0, _C)], gv[0]),
                 (glb.at[pl.ds(row0, _C)], gv[1]),
                 (gii.at[pl.ds(row0, _C)], gv[2]),
                 (gli.at[pl.ds(row0, _C)], gv[3]),
                 (ids_a.at[pl.ds(row0, _C)], gv[4]),
                 (ids_b.at[pl.ds(row0, _C)], gv[5])]
        if t == 0:
            pairs += [(labels, lab_v), (alabels, alab_v)]
        return pairs, gv

    for t in (0, 1):
        pairs, gv = chunk_pairs(t)
        for s, d in pairs:
            pltpu.async_copy(s, d, gv[7])

    zero16 = jnp.zeros((16,), jnp.int32)
    for t in range(_NCH):
        row0 = base + t * _C
        pairs, gv = chunk_pairs(t)
        gib_v, glb_v, gii_v, gli_v, ia_v, ib_v, out_v, ld, st = gv
        for s, d in pairs:
            pltpu.make_async_copy(s, d, ld).wait()
        if t >= 2:
            pltpu.make_async_copy(
                out_v, out.at[pl.ds(base + (t - 2) * _C, _C)], st).wait()
        for r in range(_C):
            rsplat = jnp.full((16,), r, jnp.int32)
            lab_i = plsc.load_gather(
                lab_v, [zero16, jnp.full((16,), row0 + r, jnp.int32)])
            for c in range(_JPAD // 16):
                s = c * 16
                jb = ib_v[r, pl.ds(s, 16)]
                ja = ia_v[r, pl.ds(s, 16)]
                pen_b = jnp.where(
                    plsc.load_gather(alab_v, [zero16, jb]) == lab_i,
                    NEG_INF, 0.0)
                pen_a = jnp.where(
                    plsc.load_gather(lab_v, [zero16, ja]) == lab_i,
                    NEG_INF, 0.0)
                out_v[r, pl.ds(s, 16)] = (
                    plsc.load_gather(gib_v, [rsplat, jb]) + pen_b)
                out_v[r, pl.ds(_JPAD + s, 16)] = (
                    plsc.load_gather(gli_v, [rsplat, ja]) + pen_a)
                out_v[r, pl.ds(2 * _JPAD + s, 16)] = (
                    plsc.load_gather(glb_v, [rsplat, jb]) + pen_b)
                out_v[r, pl.ds(3 * _JPAD + s, 16)] = (
                    plsc.load_gather(gii_v, [rsplat, ja]) + pen_a)
        pltpu.async_copy(out_v, out.at[pl.ds(row0, _C)], st)
        if t + 2 < _NCH:
            npairs, ngv = chunk_pairs(t + 2)
            for s, d in npairs:
                pltpu.async_copy(s, d, ngv[7])
    for t in (_NCH - 2, _NCH - 1):
        gv = bufs[t % 2]
        pltpu.make_async_copy(
            gv[6], out.at[pl.ds(base + t * _C, _C)], gv[8]).wait()


@functools.lru_cache(maxsize=1)
def _sc_gather():
    return pl.kernel(
        _sc_body,
        out_type=jax.ShapeDtypeStruct((_B, _W), jnp.float32),
        mesh=plsc.VectorSubcoreMesh(core_axis_name="c", subcore_axis_name="s"),
        compiler_params=pltpu.CompilerParams(needs_layout_passes=False),
        scratch_types=(
            [pltpu.VMEM((_C, _B), jnp.float32)] * 4
            + [pltpu.VMEM((_C, _JPAD), jnp.int32)] * 2
            + [pltpu.VMEM((_C, _W), jnp.float32)]
            + [pltpu.VMEM((_C, _B), jnp.float32)] * 4
            + [pltpu.VMEM((_C, _JPAD), jnp.int32)] * 2
            + [pltpu.VMEM((_C, _W), jnp.float32)]
            + [pltpu.VMEM((1, _B), jnp.float32)] * 2
            + [pltpu.SemaphoreType.DMA] * 4
        ),
    )


def _loss_body(sims_ref, sp_ref, loss_ref, acc_ref):
    x = sims_ref[...]
    sp = sp_ref[...][:, 0]
    col = lax.broadcasted_iota(jnp.int32, (_B, _W), 1)
    jj = col % _JPAD
    seg = col // _JPAD
    valid = jj < NUM_NEG
    xm = jnp.where(valid, x, NEG_INF)
    softm = valid & (seg < 2)
    xs = jnp.where(softm, xm, NEG_INF)
    m = jnp.maximum(jnp.max(xs, axis=1), sp)
    ssum = (jnp.sum(jnp.where(softm, jnp.exp(xs - m[:, None]), 0.0), axis=1)
            + jnp.exp(sp - m))
    softmax_loss = m + jnp.log(ssum) - sp
    ce_neg = jnp.where(
        valid, jnp.maximum(xm, 0.0) + jnp.log1p(jnp.exp(-jnp.abs(xm))), 0.0)
    ce_pos = jnp.maximum(sp, 0.0) - sp + jnp.log1p(jnp.exp(-jnp.abs(sp)))
    sigmoid_loss = (jnp.sum(ce_neg, axis=1) + ce_pos) / (4 * NUM_NEG + 1)
    negmax = jnp.max(jnp.where(valid & (seg == 0), xm, NEG_INF), axis=1)
    sim_max = jnp.maximum(sp, negmax)
    acc_ref[...] = jnp.mean((sim_max == sp).astype(jnp.float32)).reshape(1, 1)
    loss_ref[...] = jnp.mean(softmax_loss + sigmoid_loss).reshape(1, 1)


_loss_call = pl.pallas_call(
    _loss_body,
    out_shape=[
        jax.ShapeDtypeStruct((1, 1), jnp.float32),
        jax.ShapeDtypeStruct((1, 1), jnp.float32),
    ],
)


def kernel(inputs_embed, labels_embed, labels, all_labels_embed, all_labels):
    i = inputs_embed.astype(jnp.float32)
    l = labels_embed.astype(jnp.float32)
    bp = jnp.zeros((_B, _D), jnp.float32).at[:_NL].set(all_labels_embed)
    gib, glb, gii, gli, sp = _mm_call(i, l, i, bp)
    lab1 = labels[:, 0].astype(jnp.float32).reshape(1, _B)
    alab1 = jnp.concatenate(
        [all_labels[:, 0].astype(jnp.float32),
         jnp.full((_B - _NL,), -1.0, jnp.float32)]).reshape(1, _B)
    ids_a, ids_b = _neg_ids()
    sims = _sc_gather()(gib, glb, gii, gli, lab1, alab1, ids_a, ids_b)
    loss, acc = _loss_call(sims, sp)
    return loss[0, 0], acc[0, 0]


# speedup vs baseline: 20.7975x; 1.6414x over previous
import functools

import jax
import jax.numpy as jnp
import numpy as np
from jax import lax
from jax.experimental import pallas as pl
from jax.experimental.pallas import tpu as pltpu
from jax.experimental.pallas import tpu_sc as plsc

NUM_NEG = 50
NEG_INF = -1e9
_B = 1024
_D = 128
_NL = 1000
_JPAD = 64
_W = 4 * _JPAD

def _draw_neg_ids():
    with jax.default_device(jax.devices("cpu")[0]):
        ka, kb = jax.random.split(jax.random.key(42))
        a = np.asarray(jax.random.randint(ka, (_B, NUM_NEG), 0, _B), np.int32)
        b = np.asarray(jax.random.randint(kb, (_B, NUM_NEG), 0, _NL), np.int32)
    ids_a = np.zeros((_B, _JPAD), np.int32)
    ids_b = np.zeros((_B, _JPAD), np.int32)
    ids_a[:, :NUM_NEG] = a
    ids_b[:, :NUM_NEG] = b
    return ids_a, ids_b


try:
    _IDS_A, _IDS_B = _draw_neg_ids()
except Exception:
    _IDS_A = _IDS_B = None


def _neg_ids():
    if _IDS_A is not None:
        return jnp.asarray(_IDS_A), jnp.asarray(_IDS_B)
    ka, kb = jax.random.split(jax.random.key(42))
    a = jax.random.randint(ka, (_B, NUM_NEG), 0, _B).astype(jnp.int32)
    b = jax.random.randint(kb, (_B, NUM_NEG), 0, _NL).astype(jnp.int32)
    pad = jnp.zeros((_B, _JPAD - NUM_NEG), jnp.int32)
    return (jnp.concatenate([a, pad], axis=1),
            jnp.concatenate([b, pad], axis=1))

_RB = 256


def _mm_body(i_blk, l_blk, i_full, b_full, gib, glb, gii, gli, sp):
    ib = i_blk[...]
    lb = l_blk[...]
    it = i_full[...]
    bt = b_full[...]
    dot_hi = functools.partial(
        lax.dot_general,
        dimension_numbers=(((1,), (1,)), ((), ())),
        precision=lax.Precision.HIGHEST,
        preferred_element_type=jnp.float32,
    )
    dot_lo = functools.partial(
        lax.dot_general,
        dimension_numbers=(((1,), (1,)), ((), ())),
        precision=lax.Precision.DEFAULT,
        preferred_element_type=jnp.float32,
    )
    gib[...] = dot_hi(ib, bt)
    glb[...] = dot_lo(lb, bt)
    gii[...] = dot_lo(ib, it)
    gli[...] = dot_lo(lb, it)
    sp[...] = jnp.sum(ib * lb, axis=1, keepdims=True)


_mm_call = pl.pallas_call(
    _mm_body,
    grid=(_B // _RB,),
    in_specs=[
        pl.BlockSpec((_RB, _D), lambda g: (g, 0)),
        pl.BlockSpec((_RB, _D), lambda g: (g, 0)),
        pl.BlockSpec((_B, _D), lambda g: (0, 0)),
        pl.BlockSpec((_B, _D), lambda g: (0, 0)),
    ],
    out_specs=[
        pl.BlockSpec((_RB, _B), lambda g: (g, 0)),
        pl.BlockSpec((_RB, _B), lambda g: (g, 0)),
        pl.BlockSpec((_RB, _B), lambda g: (g, 0)),
        pl.BlockSpec((_RB, _B), lambda g: (g, 0)),
        pl.BlockSpec((_RB, 1), lambda g: (g, 0)),
    ],
    out_shape=[
        jax.ShapeDtypeStruct((_B, _B), jnp.float32),
        jax.ShapeDtypeStruct((_B, _B), jnp.float32),
        jax.ShapeDtypeStruct((_B, _B), jnp.float32),
        jax.ShapeDtypeStruct((_B, _B), jnp.float32),
        jax.ShapeDtypeStruct((_B, 1), jnp.float32),
    ],
)

_NW = 32
_RPW = _B // _NW
_C = 8
_NCH = _RPW // _C


def _sc_body(gib, glb, gii, gli, labels, alabels, ids_a, ids_b, out,
             gib0, glb0, gii0, gli0, ia0, ib0, out0,
             gib1, glb1, gii1, gli1, ia1, ib1, out1,
             lab_v, alab_v, ld0, ld1, st0, st1):
    wid = lax.axis_index("c") * 16 + lax.axis_index("s")
    base = wid * _RPW
    bufs = [(gib0, glb0, gii0, gli0, ia0, ib0, out0, ld0, st0),
            (gib1, glb1, gii1, gli1, ia1, ib1, out1, ld1, st1)]

    def chunk_pairs(t):
        row0 = base + t * _C
        gv = bufs[t % 2]
        pairs = [(gib.at[pl.ds(row0, _C)], gv[0]),
                 (glb.at[pl.ds(row0, _C)], gv[1]),
                 (gii.at[pl.ds(row0, _C)], gv[2]),
                 (gli.at[pl.ds(row0, _C)], gv[3]),
                 (ids_a.at[pl.ds(row0, _C)], gv[4]),
                 (ids_b.at[pl.ds(row0, _C)], gv[5])]
        if t == 0:
            pairs += [(labels, lab_v), (alabels, alab_v)]
        return pairs, gv

    for t in (0, 1):
        pairs, gv = chunk_pairs(t)
        for s, d in pairs:
            pltpu.async_copy(s, d, gv[7])

    zero16 = jnp.zeros((16,), jnp.int32)
    for t in range(_NCH):
        row0 = base + t * _C
        pairs, gv = chunk_pairs(t)
        gib_v, glb_v, gii_v, gli_v, ia_v, ib_v, out_v, ld, st = gv
        for s, d in pairs:
            pltpu.make_async_copy(s, d, ld).wait()
        if t >= 2:
            pltpu.make_async_copy(
                out_v, out.at[pl.ds(base + (t - 2) * _C, _C)], st).wait()
        for r in range(_C):
            rsplat = jnp.full((16,), r, jnp.int32)
            lab_i = plsc.load_gather(
                lab_v, [zero16, jnp.full((16,), row0 + r, jnp.int32)])
            for c in range(_JPAD // 16):
                s = c * 16
                jb = ib_v[r, pl.ds(s, 16)]
                ja = ia_v[r, pl.ds(s, 16)]
                pen_b = jnp.where(
                    plsc.load_gather(alab_v, [zero16, jb]) == lab_i,
                    NEG_INF, 0.0)
                pen_a = jnp.where(
                    plsc.load_gather(lab_v, [zero16, ja]) == lab_i,
                    NEG_INF, 0.0)
                out_v[r, pl.ds(s, 16)] = (
                    plsc.load_gather(gib_v, [rsplat, jb]) + pen_b)
                out_v[r, pl.ds(_JPAD + s, 16)] = (
                    plsc.load_gather(gli_v, [rsplat, ja]) + pen_a)
                out_v[r, pl.ds(2 * _JPAD + s, 16)] = (
                    plsc.load_gather(glb_v, [rsplat, jb]) + pen_b)
                out_v[r, pl.ds(3 * _JPAD + s, 16)] = (
                    plsc.load_gather(gii_v, [rsplat, ja]) + pen_a)
        pltpu.async_copy(out_v, out.at[pl.ds(row0, _C)], st)
        if t + 2 < _NCH:
            npairs, ngv = chunk_pairs(t + 2)
            for s, d in npairs:
                pltpu.async_copy(s, d, ngv[7])
    for t in (_NCH - 2, _NCH - 1):
        gv = bufs[t % 2]
        pltpu.make_async_copy(
            gv[6], out.at[pl.ds(base + t * _C, _C)], gv[8]).wait()


@functools.lru_cache(maxsize=1)
def _sc_gather():
    return pl.kernel(
        _sc_body,
        out_type=jax.ShapeDtypeStruct((_B, _W), jnp.float32),
        mesh=plsc.VectorSubcoreMesh(core_axis_name="c", subcore_axis_name="s"),
        compiler_params=pltpu.CompilerParams(needs_layout_passes=False),
        scratch_types=(
            [pltpu.VMEM((_C, _B), jnp.float32)] * 4
            + [pltpu.VMEM((_C, _JPAD), jnp.int32)] * 2
            + [pltpu.VMEM((_C, _W), jnp.float32)]
            + [pltpu.VMEM((_C, _B), jnp.float32)] * 4
            + [pltpu.VMEM((_C, _JPAD), jnp.int32)] * 2
            + [pltpu.VMEM((_C, _W), jnp.float32)]
            + [pltpu.VMEM((1, _B), jnp.float32)] * 2
            + [pltpu.SemaphoreType.DMA] * 4
        ),
    )


def _loss_body(sims_ref, sp_ref, loss_ref, acc_ref):
    x = sims_ref[...]
    sp = sp_ref[...][:, 0]
    col = lax.broadcasted_iota(jnp.int32, (_B, _W), 1)
    jj = col % _JPAD
    seg = col // _JPAD
    valid = jj < NUM_NEG
    xm = jnp.where(valid, x, NEG_INF)
    softm = valid & (seg < 2)
    xs = jnp.where(softm, xm, NEG_INF)
    m = jnp.maximum(jnp.max(xs, axis=1), sp)
    ssum = (jnp.sum(jnp.where(softm, jnp.exp(xs - m[:, None]), 0.0), axis=1)
            + jnp.exp(sp - m))
    softmax_loss = m + jnp.log(ssum) - sp
    ce_neg = jnp.where(
        valid, jnp.maximum(xm, 0.0) + jnp.log1p(jnp.exp(-jnp.abs(xm))), 0.0)
    ce_pos = jnp.maximum(sp, 0.0) - sp + jnp.log1p(jnp.exp(-jnp.abs(sp)))
    sigmoid_loss = (jnp.sum(ce_neg, axis=1) + ce_pos) / (4 * NUM_NEG + 1)
    negmax = jnp.max(jnp.where(valid & (seg == 0), xm, NEG_INF), axis=1)
    sim_max = jnp.maximum(sp, negmax)
    acc_ref[...] = jnp.mean((sim_max == sp).astype(jnp.float32)).reshape(1, 1)
    loss_ref[...] = jnp.mean(softmax_loss + sigmoid_loss).reshape(1, 1)


_loss_call = pl.pallas_call(
    _loss_body,
    out_shape=[
        jax.ShapeDtypeStruct((1, 1), jnp.float32),
        jax.ShapeDtypeStruct((1, 1), jnp.float32),
    ],
)


def kernel(inputs_embed, labels_embed, labels, all_labels_embed, all_labels):
    i = inputs_embed.astype(jnp.float32)
    l = labels_embed.astype(jnp.float32)
    bp = jnp.zeros((_B, _D), jnp.float32).at[:_NL].set(all_labels_embed)
    gib, glb, gii, gli, sp = _mm_call(i, l, i, bp)
    lab1 = labels[:, 0].astype(jnp.float32).reshape(1, _B)
    alab1 = jnp.concatenate(
        [all_labels[:, 0].astype(jnp.float32),
         jnp.full((_B - _NL,), -1.0, jnp.float32)]).reshape(1, _B)
    ids_a, ids_b = _neg_ids()
    sims = _sc_gather()(gib, glb, gii, gli, lab1, alab1, ids_a, ids_b)
    loss, acc = _loss_call(sims, sp)
    return loss[0, 0], acc[0, 0]


# speedup vs baseline: 22.3606x; 1.0752x over previous
import functools

import jax
import jax.numpy as jnp
import numpy as np
from jax import lax
from jax.experimental import pallas as pl
from jax.experimental.pallas import tpu as pltpu
from jax.experimental.pallas import tpu_sc as plsc

NUM_NEG = 50
NEG_INF = -1e9
_B = 1024
_D = 128
_NL = 1000
_JPAD = 64
_W = 4 * _JPAD

def _draw_neg_ids():
    with jax.default_device(jax.devices("cpu")[0]):
        ka, kb = jax.random.split(jax.random.key(42))
        a = np.asarray(jax.random.randint(ka, (_B, NUM_NEG), 0, _B), np.int32)
        b = np.asarray(jax.random.randint(kb, (_B, NUM_NEG), 0, _NL), np.int32)
    ids_a = np.zeros((_B, _JPAD), np.int32)
    ids_b = np.zeros((_B, _JPAD), np.int32)
    ids_a[:, :NUM_NEG] = a
    ids_b[:, :NUM_NEG] = b
    return ids_a, ids_b


try:
    _IDS_A, _IDS_B = _draw_neg_ids()
except Exception:
    _IDS_A = _IDS_B = None


def _neg_ids():
    if _IDS_A is not None:
        return jnp.asarray(_IDS_A), jnp.asarray(_IDS_B)
    ka, kb = jax.random.split(jax.random.key(42))
    a = jax.random.randint(ka, (_B, NUM_NEG), 0, _B).astype(jnp.int32)
    b = jax.random.randint(kb, (_B, NUM_NEG), 0, _NL).astype(jnp.int32)
    pad = jnp.zeros((_B, _JPAD - NUM_NEG), jnp.int32)
    return (jnp.concatenate([a, pad], axis=1),
            jnp.concatenate([b, pad], axis=1))

_RB = 256


_HW = _B // 2


def _pack16(lo, hi):
    blo = lax.bitcast_convert_type(lo, jnp.int32) + jnp.int32(0x8000)
    bhi = lax.bitcast_convert_type(hi, jnp.int32) + jnp.int32(0x8000)
    return (bhi & jnp.int32(-65536)) | lax.shift_right_logical(blo, 16)


def _mm_body(i_blk, l_blk, i_full, b_full, gib, glb, gii, gli, sp):
    ib = i_blk[...]
    lb = l_blk[...]
    it = i_full[...]
    bt = b_full[...]
    dot_hi = functools.partial(
        lax.dot_general,
        dimension_numbers=(((1,), (1,)), ((), ())),
        precision=lax.Precision.HIGHEST,
        preferred_element_type=jnp.float32,
    )
    dot_lo = functools.partial(
        lax.dot_general,
        dimension_numbers=(((1,), (1,)), ((), ())),
        precision=lax.Precision.DEFAULT,
        preferred_element_type=jnp.float32,
    )
    gib[...] = dot_hi(ib, bt)
    g = dot_lo(lb, bt)
    glb[...] = _pack16(g[:, :_HW], g[:, _HW:])
    g = dot_lo(ib, it)
    gii[...] = _pack16(g[:, :_HW], g[:, _HW:])
    g = dot_lo(lb, it)
    gli[...] = _pack16(g[:, :_HW], g[:, _HW:])
    sp[...] = jnp.sum(ib * lb, axis=1, keepdims=True)


_mm_call = pl.pallas_call(
    _mm_body,
    grid=(_B // _RB,),
    in_specs=[
        pl.BlockSpec((_RB, _D), lambda g: (g, 0)),
        pl.BlockSpec((_RB, _D), lambda g: (g, 0)),
        pl.BlockSpec((_B, _D), lambda g: (0, 0)),
        pl.BlockSpec((_B, _D), lambda g: (0, 0)),
    ],
    out_specs=[
        pl.BlockSpec((_RB, _B), lambda g: (g, 0)),
        pl.BlockSpec((_RB, _HW), lambda g: (g, 0)),
        pl.BlockSpec((_RB, _HW), lambda g: (g, 0)),
        pl.BlockSpec((_RB, _HW), lambda g: (g, 0)),
        pl.BlockSpec((_RB, 1), lambda g: (g, 0)),
    ],
    out_shape=[
        jax.ShapeDtypeStruct((_B, _B), jnp.float32),
        jax.ShapeDtypeStruct((_B, _HW), jnp.int32),
        jax.ShapeDtypeStruct((_B, _HW), jnp.int32),
        jax.ShapeDtypeStruct((_B, _HW), jnp.int32),
        jax.ShapeDtypeStruct((_B, 1), jnp.float32),
    ],
)

_NW = 32
_RPW = _B // _NW
_C = 8
_NCH = _RPW // _C


def _unpack16(ref, rsplat, j):
    sel = j >= _HW
    widx = jnp.where(sel, j - _HW, j)
    w = plsc.load_gather(ref, [rsplat, widx])
    bits = jnp.where(sel, w, w << 16) & jnp.int32(-65536)
    return plsc.bitcast(bits, jnp.float32)


def _sc_body(gib, glb, gii, gli, labels, alabels, ids_a, ids_b, out,
             gib0, glb0, gii0, gli0, ia0, ib0, out0,
             gib1, glb1, gii1, gli1, ia1, ib1, out1,
             lab_v, alab_v, ld0, ld1, st0, st1):
    wid = lax.axis_index("c") * 16 + lax.axis_index("s")
    base = wid * _RPW
    bufs = [(gib0, glb0, gii0, gli0, ia0, ib0, out0, ld0, st0),
            (gib1, glb1, gii1, gli1, ia1, ib1, out1, ld1, st1)]

    def chunk_pairs(t):
        row0 = base + t * _C
        gv = bufs[t % 2]
        pairs = [(gib.at[pl.ds(row0, _C)], gv[0]),
                 (glb.at[pl.ds(row0, _C)], gv[1]),
                 (gii.at[pl.ds(row0, _C)], gv[2]),
                 (gli.at[pl.ds(row0, _C)], gv[3]),
                 (ids_a.at[pl.ds(row0, _C)], gv[4]),
                 (ids_b.at[pl.ds(row0, _C)], gv[5])]
        if t == 0:
            pairs += [(labels, lab_v), (alabels, alab_v)]
        return pairs, gv

    for t in (0, 1):
        pairs, gv = chunk_pairs(t)
        for s, d in pairs:
            pltpu.async_copy(s, d, gv[7])

    zero16 = jnp.zeros((16,), jnp.int32)
    for t in range(_NCH):
        row0 = base + t * _C
        pairs, gv = chunk_pairs(t)
        gib_v, glb_v, gii_v, gli_v, ia_v, ib_v, out_v, ld, st = gv
        for s, d in pairs:
            pltpu.make_async_copy(s, d, ld).wait()
        if t >= 2:
            pltpu.make_async_copy(
                out_v, out.at[pl.ds(base + (t - 2) * _C, _C)], st).wait()
        for r in range(_C):
            rsplat = jnp.full((16,), r, jnp.int32)
            lab_i = plsc.load_gather(
                lab_v, [zero16, jnp.full((16,), row0 + r, jnp.int32)])
            for c in range(_JPAD // 16):
                s = c * 16
                jb = ib_v[r, pl.ds(s, 16)]
                ja = ia_v[r, pl.ds(s, 16)]
                pen_b = jnp.where(
                    plsc.load_gather(alab_v, [zero16, jb]) == lab_i,
                    NEG_INF, 0.0)
                pen_a = jnp.where(
                    plsc.load_gather(lab_v, [zero16, ja]) == lab_i,
                    NEG_INF, 0.0)
                out_v[r, pl.ds(s, 16)] = (
                    plsc.load_gather(gib_v, [rsplat, jb]) + pen_b)
                out_v[r, pl.ds(_JPAD + s, 16)] = (
                    _unpack16(gli_v, rsplat, ja) + pen_a)
                out_v[r, pl.ds(2 * _JPAD + s, 16)] = (
                    _unpack16(glb_v, rsplat, jb) + pen_b)
                out_v[r, pl.ds(3 * _JPAD + s, 16)] = (
                    _unpack16(gii_v, rsplat, ja) + pen_a)
        pltpu.async_copy(out_v, out.at[pl.ds(row0, _C)], st)
        if t + 2 < _NCH:
            npairs, ngv = chunk_pairs(t + 2)
            for s, d in npairs:
                pltpu.async_copy(s, d, ngv[7])
    for t in (_NCH - 2, _NCH - 1):
        gv = bufs[t % 2]
        pltpu.make_async_copy(
            gv[6], out.at[pl.ds(base + t * _C, _C)], gv[8]).wait()


@functools.lru_cache(maxsize=1)
def _sc_gather():
    return pl.kernel(
        _sc_body,
        out_type=jax.ShapeDtypeStruct((_B, _W), jnp.float32),
        mesh=plsc.VectorSubcoreMesh(core_axis_name="c", subcore_axis_name="s"),
        compiler_params=pltpu.CompilerParams(needs_layout_passes=False),
        scratch_types=(
            [pltpu.VMEM((_C, _B), jnp.float32)]
            + [pltpu.VMEM((_C, _HW), jnp.int32)] * 3
            + [pltpu.VMEM((_C, _JPAD), jnp.int32)] * 2
            + [pltpu.VMEM((_C, _W), jnp.float32)]
            + [pltpu.VMEM((_C, _B), jnp.float32)]
            + [pltpu.VMEM((_C, _HW), jnp.int32)] * 3
            + [pltpu.VMEM((_C, _JPAD), jnp.int32)] * 2
            + [pltpu.VMEM((_C, _W), jnp.float32)]
            + [pltpu.VMEM((1, _B), jnp.float32)] * 2
            + [pltpu.SemaphoreType.DMA] * 4
        ),
    )


def _loss_body(sims_ref, sp_ref, loss_ref, acc_ref):
    x = sims_ref[...]
    sp = sp_ref[...][:, 0]
    col = lax.broadcasted_iota(jnp.int32, (_B, _W), 1)
    jj = col % _JPAD
    seg = col // _JPAD
    valid = jj < NUM_NEG
    xm = jnp.where(valid, x, NEG_INF)
    softm = valid & (seg < 2)
    xs = jnp.where(softm, xm, NEG_INF)
    m = jnp.maximum(jnp.max(xs, axis=1), sp)
    ssum = (jnp.sum(jnp.where(softm, jnp.exp(xs - m[:, None]), 0.0), axis=1)
            + jnp.exp(sp - m))
    softmax_loss = m + jnp.log(ssum) - sp
    ce_neg = jnp.where(
        valid, jnp.maximum(xm, 0.0) + jnp.log1p(jnp.exp(-jnp.abs(xm))), 0.0)
    ce_pos = jnp.maximum(sp, 0.0) - sp + jnp.log1p(jnp.exp(-jnp.abs(sp)))
    sigmoid_loss = (jnp.sum(ce_neg, axis=1) + ce_pos) / (4 * NUM_NEG + 1)
    negmax = jnp.max(jnp.where(valid & (seg == 0), xm, NEG_INF), axis=1)
    sim_max = jnp.maximum(sp, negmax)
    acc_ref[...] = jnp.mean((sim_max == sp).astype(jnp.float32)).reshape(1, 1)
    loss_ref[...] = jnp.mean(softmax_loss + sigmoid_loss).reshape(1, 1)


_loss_call = pl.pallas_call(
    _loss_body,
    out_shape=[
        jax.ShapeDtypeStruct((1, 1), jnp.float32),
        jax.ShapeDtypeStruct((1, 1), jnp.float32),
    ],
)


def kernel(inputs_embed, labels_embed, labels, all_labels_embed, all_labels):
    i = inputs_embed.astype(jnp.float32)
    l = labels_embed.astype(jnp.float32)
    bp = jnp.zeros((_B, _D), jnp.float32).at[:_NL].set(all_labels_embed)
    gib, glb, gii, gli, sp = _mm_call(i, l, i, bp)
    lab1 = labels[:, 0].astype(jnp.float32).reshape(1, _B)
    alab1 = jnp.concatenate(
        [all_labels[:, 0].astype(jnp.float32),
         jnp.full((_B - _NL,), -1.0, jnp.float32)]).reshape(1, _B)
    ids_a, ids_b = _neg_ids()
    sims = _sc_gather()(gib, glb, gii, gli, lab1, alab1, ids_a, ids_b)
    loss, acc = _loss_call(sims, sp)
    return loss[0, 0], acc[0, 0]
